# baseline (speedup 1.0000x reference)
import functools

import jax
import jax.numpy as jnp
from jax import lax
from jax.experimental import pallas as pl
from jax.experimental.pallas import tpu as pltpu
from jax.experimental.pallas import tpu_sc as plsc

NUM_VOCAB = 1000000
NUM_EMBED = 64
NUM_TOKEN = 200
BATCH = 1024

NW = 32
B_PER_W = BATCH // NW
CHUNK_A = 96
CHUNK_B = 104
LANES = 16
C_PER_ROW = NUM_EMBED // LANES


def _emb_kernel(tok_hbm, table_hbm, pos_hbm, out_hbm, idx_v, pos_v, rows_v,
                gsem):
  wid = lax.axis_index("s") * 2 + lax.axis_index("c")
  b0 = wid * B_PER_W

  pltpu.sync_copy(tok_hbm.at[pl.ds(b0, B_PER_W)], idx_v)
  pltpu.sync_copy(pos_hbm, pos_v)

  def batch_body(p, carry):
    cp0 = pltpu.async_copy(table_hbm.at[idx_v.at[p, pl.ds(0, CHUNK_A)]],
                           rows_v.at[pl.ds(0, CHUNK_A)], gsem)
    cp1 = pltpu.async_copy(table_hbm.at[idx_v.at[p, pl.ds(CHUNK_A, CHUNK_B)]],
                           rows_v.at[pl.ds(CHUNK_A, CHUNK_B)], gsem)
    cp0.wait()
    cp1.wait()

    def row_body(t, c2):
      for c in range(C_PER_ROW):
        sl = pl.ds(c * LANES, LANES)
        rows_v[t, sl] = rows_v[t, sl] + pos_v[t, sl]
      return c2

    lax.fori_loop(0, NUM_TOKEN, row_body, 0, unroll=4)
    pltpu.sync_copy(rows_v, out_hbm.at[b0 + p])
    return carry

  lax.fori_loop(0, B_PER_W, batch_body, 0)


@jax.jit
def _emb(tokens, table, positionembed):
  mesh = plsc.VectorSubcoreMesh(core_axis_name="c", subcore_axis_name="s")
  run = functools.partial(
      pl.kernel,
      mesh=mesh,
      compiler_params=pltpu.CompilerParams(use_tc_tiling_on_sc=False),
      out_type=jax.ShapeDtypeStruct((BATCH, NUM_TOKEN, NUM_EMBED),
                                    jnp.float32),
      scratch_types=[
          pltpu.VMEM((B_PER_W, NUM_TOKEN), jnp.int32),
          pltpu.VMEM((NUM_TOKEN, NUM_EMBED), jnp.float32),
          pltpu.VMEM((NUM_TOKEN, NUM_EMBED), jnp.float32),
          pltpu.SemaphoreType.DMA,
      ],
  )(_emb_kernel)
  return run(tokens, table, positionembed)


def kernel(tokens, table, positionembed):
  return _emb(tokens.astype(jnp.int32), table, positionembed)


# speedup vs baseline: 1.1002x; 1.1002x over previous
import functools

import jax
import jax.numpy as jnp
from jax import lax
from jax.experimental import pallas as pl
from jax.experimental.pallas import tpu as pltpu
from jax.experimental.pallas import tpu_sc as plsc

NUM_VOCAB = 1000000
NUM_EMBED = 64
NUM_TOKEN = 200
BATCH = 1024
TOK_PAD = 256

NW = 32
B_PER_W = BATCH // NW
B_TOTAL = BATCH * NUM_TOKEN
CHUNK_A = 96
CHUNK_B = 104
LANES = 16
C_PER_ROW = NUM_EMBED // LANES


def _emb_kernel(tok_hbm, table_hbm, pos_hbm, out_hbm, idx_a, idx_b, pos_v,
                rows_v, gsem):
  wid = lax.axis_index("s") * 2 + lax.axis_index("c")
  b0 = wid * B_PER_W

  pltpu.sync_copy(tok_hbm.at[pl.ds(b0, B_PER_W), pl.ds(0, CHUNK_A)], idx_a)
  pltpu.sync_copy(tok_hbm.at[pl.ds(b0, B_PER_W), pl.ds(CHUNK_A, CHUNK_B)],
                  idx_b)
  pltpu.sync_copy(pos_hbm, pos_v)

  def batch_body(p, carry):
    cp0 = pltpu.async_copy(table_hbm.at[idx_a.at[p]],
                           rows_v.at[pl.ds(0, CHUNK_A)], gsem)
    cp1 = pltpu.async_copy(table_hbm.at[idx_b.at[p]],
                           rows_v.at[pl.ds(CHUNK_A, CHUNK_B)], gsem)
    cp0.wait()
    cp1.wait()

    def row_body(t, c2):
      for c in range(C_PER_ROW):
        sl = pl.ds(c * LANES, LANES)
        rows_v[t, sl] = rows_v[t, sl] + pos_v[t, sl]
      return c2

    lax.fori_loop(0, NUM_TOKEN, row_body, 0)
    pltpu.sync_copy(rows_v,
                    out_hbm.at[pl.ds((b0 + p) * NUM_TOKEN, NUM_TOKEN)])
    return carry

  lax.fori_loop(0, B_PER_W, batch_body, 0)


@jax.jit
def _emb(tokens_p, table, positionembed):
  mesh = plsc.VectorSubcoreMesh(core_axis_name="c", subcore_axis_name="s")
  run = functools.partial(
      pl.kernel,
      mesh=mesh,
      compiler_params=pltpu.CompilerParams(use_tc_tiling_on_sc=False),
      out_type=jax.ShapeDtypeStruct((B_TOTAL, NUM_EMBED), jnp.float32),
      scratch_types=[
          pltpu.VMEM((B_PER_W, CHUNK_A), jnp.int32),
          pltpu.VMEM((B_PER_W, CHUNK_B), jnp.int32),
          pltpu.VMEM((NUM_TOKEN, NUM_EMBED), jnp.float32),
          pltpu.VMEM((NUM_TOKEN, NUM_EMBED), jnp.float32),
          pltpu.SemaphoreType.DMA,
      ],
  )(_emb_kernel)
  return run(tokens_p, table, positionembed)


def kernel(tokens, table, positionembed):
  tokens_p = jnp.pad(tokens.astype(jnp.int32),
                     ((0, 0), (0, TOK_PAD - NUM_TOKEN)))
  out = _emb(tokens_p, table, positionembed)
  return out.reshape(BATCH, NUM_TOKEN, NUM_EMBED)
